# baseline (speedup 1.0000x reference)
import jax
import jax.numpy as jnp
from jax.experimental import pallas as pl
from jax.experimental.pallas import tpu as pltpu

_TB = 2048
_S = 10
_NSLOT = 1 + 2 * _S


def _dot(x, w):
    return jax.lax.dot_general(
        x.astype(jnp.bfloat16), w,
        (((1,), (0,)), ((), ())),
        preferred_element_type=jnp.float32)


def _body(x_hbm, ws0_ref, wn0_ref, ws1_ref, wn1_ref, out_ref, buf, sem):
    i = pl.program_id(0)
    nt = pl.num_programs(0) - 1
    f32 = jnp.float32
    relu = jax.nn.relu

    @pl.when(i < nt)
    def _():
        par = i % 2
        row0 = i * _TB
        for s in range(_NSLOT):
            pltpu.make_async_copy(
                x_hbm.at[pl.ds(row0, _TB), s], buf.at[par, s], sem.at[par, s]).start()

    @pl.when(i > 0)
    def _():
        par = (i - 1) % 2
        for s in range(_NSLOT):
            pltpu.make_async_copy(
                x_hbm.at[pl.ds(0, _TB), s], buf.at[par, s], sem.at[par, s]).wait()
        ws0 = ws0_ref[...].astype(jnp.bfloat16)
        wn0 = wn0_ref[...].astype(jnp.bfloat16)
        inv = f32(1.0 / _S)

        h0a = relu(_dot(buf[par, 0], ws0))
        accb = buf[par, 1]
        m1a = relu(_dot(buf[par, 1], ws0))
        m1b = relu(_dot(buf[par, 1 + _S], wn0))
        for s in range(2, _S + 1):
            accb = accb + buf[par, s]
            m1a = m1a + relu(_dot(buf[par, s], ws0))
            m1b = m1b + relu(_dot(buf[par, s + _S], wn0))
        h0b = relu(_dot(accb * inv, wn0))
        m1a = m1a * inv
        m1b = m1b * inv

        ws1 = ws1_ref[...].astype(jnp.bfloat16)
        wn1 = wn1_ref[...].astype(jnp.bfloat16)
        d1 = ws0.shape[1]
        out_ref[:, :d1] = _dot(h0a, ws1[:d1]) + _dot(h0b, ws1[d1:])
        out_ref[:, d1:] = _dot(m1a, wn1[:d1]) + _dot(m1b, wn1[d1:])


def kernel(input_features, W_self_0, W_neigh_0, W_self_1, W_neigh_1):
    n, slots, f = input_features.shape
    d1 = W_self_0.shape[1]
    d2 = W_self_1.shape[1]
    tb = _TB
    nt = n // tb
    return pl.pallas_call(
        _body,
        grid=(nt + 1,),
        in_specs=[
            pl.BlockSpec(memory_space=pl.ANY),
            pl.BlockSpec((f, d1), lambda i: (0, 0)),
            pl.BlockSpec((f, d1), lambda i: (0, 0)),
            pl.BlockSpec((2 * d1, d2), lambda i: (0, 0)),
            pl.BlockSpec((2 * d1, d2), lambda i: (0, 0)),
        ],
        out_specs=pl.BlockSpec(
            (tb, 2 * d2), lambda i: (jnp.maximum(i - 1, 0), 0)),
        out_shape=jax.ShapeDtypeStruct((n, 2 * d2), jnp.float32),
        scratch_shapes=[
            pltpu.VMEM((2, _NSLOT, tb, f), jnp.float32),
            pltpu.SemaphoreType.DMA((2, _NSLOT)),
        ],
    )(input_features, W_self_0, W_neigh_0, W_self_1, W_neigh_1)
